# baseline (speedup 1.0000x reference)
import jax
import jax.numpy as jnp
from jax.experimental import pallas as pl
from jax.experimental.pallas import tpu as pltpu

_F_BLK = 512


def _router_body(logits_ref, ct_ref):
    logits = logits_ref[...]
    t, e = logits.shape
    m = jnp.max(logits, axis=-1, keepdims=True)
    ex = jnp.exp(logits - m)
    probs = ex / jnp.sum(ex, axis=-1, keepdims=True)
    col = jax.lax.broadcasted_iota(jnp.int32, (t, e), 1)
    v1 = jnp.max(probs, axis=-1)
    i1 = jnp.min(jnp.where(probs >= v1[:, None], col, e), axis=-1)
    masked = jnp.where(col == i1[:, None], -jnp.inf, probs)
    v2 = jnp.max(masked, axis=-1)
    i2 = jnp.min(jnp.where(masked >= v2[:, None], col, e), axis=-1)
    s = v1 + v2
    wa = (v1 / s)[:, None]
    wb = (v2 / s)[:, None]
    comb = jnp.where(col == i1[:, None], wa, 0.0) + jnp.where(col == i2[:, None], wb, 0.0)
    ct_ref[...] = comb.T


def _moe_body(ids_ref, n_ref, x_ref, ct_ref, w1_ref, b1_ref, w2_ref, b2_ref, o_ref):
    i = pl.program_id(0)
    f = pl.program_id(1)

    @pl.when(jnp.logical_and(i == 0, f == 0))
    def _init():
        o_ref[...] = jnp.zeros_like(o_ref)

    @pl.when(i < n_ref[0])
    def _compute():
        x = x_ref[...]
        h = jnp.dot(x, w1_ref[0], preferred_element_type=jnp.float32)
        h = h + b1_ref[0]
        a = jax.nn.gelu(h)
        y = jnp.dot(a, w2_ref[0], preferred_element_type=jnp.float32)
        y = y + jnp.where(f == 0, 1.0, 0.0) * b2_ref[0]
        e = ids_ref[i]
        colw = ct_ref[e, :]
        o_ref[...] += colw[:, None] * y


def kernel(hidden_states, router_logits, w1, b1, w2, b2):
    t, d = hidden_states.shape
    n_e = router_logits.shape[1]
    ffn = w1.shape[2]
    n_fc = ffn // _F_BLK

    ct = pl.pallas_call(
        _router_body,
        out_shape=jax.ShapeDtypeStruct((n_e, t), jnp.float32),
    )(router_logits)

    active = jnp.any(ct > 0.0, axis=1)
    n_active = jnp.sum(active.astype(jnp.int32))
    key = jnp.where(active, jnp.arange(n_e, dtype=jnp.int32), jnp.int32(n_e))
    sorted_ids = jnp.sort(key)
    last = sorted_ids[jnp.maximum(n_active - 1, 0)]
    ids = jnp.where(jnp.arange(n_e, dtype=jnp.int32) < n_active, sorted_ids, last)
    n_arr = jnp.reshape(n_active, (1,)).astype(jnp.int32)

    b1_3 = b1[:, None, :]
    b2_3 = b2[:, None, :]

    def _f_eff(i, f, n):
        return jnp.where(i < n[0], f, n_fc - 1)

    grid_spec = pltpu.PrefetchScalarGridSpec(
        num_scalar_prefetch=2,
        grid=(n_e, n_fc),
        in_specs=[
            pl.BlockSpec((t, d), lambda i, f, ids, n: (0, 0)),
            pl.BlockSpec((n_e, t), lambda i, f, ids, n: (0, 0)),
            pl.BlockSpec((1, d, _F_BLK), lambda i, f, ids, n: (ids[i], 0, _f_eff(i, f, n))),
            pl.BlockSpec((1, 1, _F_BLK), lambda i, f, ids, n: (ids[i], 0, _f_eff(i, f, n))),
            pl.BlockSpec((1, _F_BLK, d), lambda i, f, ids, n: (ids[i], _f_eff(i, f, n), 0)),
            pl.BlockSpec((1, 1, d), lambda i, f, ids, n: (ids[i], 0, 0)),
        ],
        out_specs=pl.BlockSpec((t, d), lambda i, f, ids, n: (0, 0)),
    )

    out = pl.pallas_call(
        _moe_body,
        grid_spec=grid_spec,
        out_shape=jax.ShapeDtypeStruct((t, d), jnp.float32),
        compiler_params=pltpu.CompilerParams(
            dimension_semantics=("arbitrary", "arbitrary"),
        ),
    )(ids, n_arr, hidden_states, ct, w1, b1_3, w2, b2_3)
    return out


# speedup vs baseline: 1.2268x; 1.2268x over previous
import jax
import jax.numpy as jnp
from jax.experimental import pallas as pl
from jax.experimental.pallas import tpu as pltpu

_F_BLK = 2048


def _router_body(logits_ref, ct_ref):
    logits = logits_ref[...]
    t, e = logits.shape
    m = jnp.max(logits, axis=-1, keepdims=True)
    ex = jnp.exp(logits - m)
    probs = ex / jnp.sum(ex, axis=-1, keepdims=True)
    col = jax.lax.broadcasted_iota(jnp.int32, (t, e), 1)
    v1 = jnp.max(probs, axis=-1)
    i1 = jnp.min(jnp.where(probs >= v1[:, None], col, e), axis=-1)
    masked = jnp.where(col == i1[:, None], -jnp.inf, probs)
    v2 = jnp.max(masked, axis=-1)
    i2 = jnp.min(jnp.where(masked >= v2[:, None], col, e), axis=-1)
    s = v1 + v2
    wa = (v1 / s)[:, None]
    wb = (v2 / s)[:, None]
    comb = jnp.where(col == i1[:, None], wa, 0.0) + jnp.where(col == i2[:, None], wb, 0.0)
    ct_ref[...] = comb.T


def _moe_body(ids_ref, n_ref, x_ref, ct_ref, w1_ref, b1_ref, w2_ref, b2_ref, o_ref):
    i = pl.program_id(0)
    f = pl.program_id(1)

    @pl.when(jnp.logical_and(i == 0, f == 0))
    def _init():
        o_ref[...] = jnp.zeros_like(o_ref)

    @pl.when(i < n_ref[0])
    def _compute():
        x = x_ref[...]
        h = jnp.dot(x, w1_ref[0], preferred_element_type=jnp.float32)
        h = h + b1_ref[0]
        a = jax.nn.gelu(h)
        y = jnp.dot(a, w2_ref[0], preferred_element_type=jnp.float32)
        y = y + jnp.where(f == 0, 1.0, 0.0) * b2_ref[0]
        e = ids_ref[i]
        colw = ct_ref[e, :]
        o_ref[...] += colw[:, None] * y


def kernel(hidden_states, router_logits, w1, b1, w2, b2):
    t, d = hidden_states.shape
    n_e = router_logits.shape[1]
    ffn = w1.shape[2]
    n_fc = ffn // _F_BLK

    ct = pl.pallas_call(
        _router_body,
        out_shape=jax.ShapeDtypeStruct((n_e, t), jnp.float32),
    )(router_logits)

    active = jnp.any(ct > 0.0, axis=1)
    n_active = jnp.sum(active.astype(jnp.int32))
    key = jnp.where(active, jnp.arange(n_e, dtype=jnp.int32), jnp.int32(n_e))
    sorted_ids = jnp.sort(key)
    last = sorted_ids[jnp.maximum(n_active - 1, 0)]
    ids = jnp.where(jnp.arange(n_e, dtype=jnp.int32) < n_active, sorted_ids, last)
    n_arr = jnp.reshape(n_active, (1,)).astype(jnp.int32)

    b1_3 = b1[:, None, :]
    b2_3 = b2[:, None, :]

    def _f_eff(i, f, n):
        return jnp.where(i < n[0], f, n_fc - 1)

    grid_spec = pltpu.PrefetchScalarGridSpec(
        num_scalar_prefetch=2,
        grid=(n_e, n_fc),
        in_specs=[
            pl.BlockSpec((t, d), lambda i, f, ids, n: (0, 0)),
            pl.BlockSpec((n_e, t), lambda i, f, ids, n: (0, 0)),
            pl.BlockSpec((1, d, _F_BLK), lambda i, f, ids, n: (ids[i], 0, _f_eff(i, f, n))),
            pl.BlockSpec((1, 1, _F_BLK), lambda i, f, ids, n: (ids[i], 0, _f_eff(i, f, n))),
            pl.BlockSpec((1, _F_BLK, d), lambda i, f, ids, n: (ids[i], _f_eff(i, f, n), 0)),
            pl.BlockSpec((1, 1, d), lambda i, f, ids, n: (ids[i], 0, 0)),
        ],
        out_specs=pl.BlockSpec((t, d), lambda i, f, ids, n: (0, 0)),
    )

    out = pl.pallas_call(
        _moe_body,
        grid_spec=grid_spec,
        out_shape=jax.ShapeDtypeStruct((t, d), jnp.float32),
        compiler_params=pltpu.CompilerParams(
            dimension_semantics=("arbitrary", "arbitrary"),
        ),
    )(ids, n_arr, hidden_states, ct, w1, b1_3, w2, b2_3)
    return out


# speedup vs baseline: 1.2585x; 1.0258x over previous
import jax
import jax.numpy as jnp
from jax.experimental import pallas as pl
from jax.experimental.pallas import tpu as pltpu

_F_BLK = 2048


def _router_body(logits_ref, ct_ref, meta_ref):
    logits = logits_ref[...]
    t, e = logits.shape
    m = jnp.max(logits, axis=-1, keepdims=True)
    ex = jnp.exp(logits - m)
    probs = ex / jnp.sum(ex, axis=-1, keepdims=True)
    col = jax.lax.broadcasted_iota(jnp.int32, (t, e), 1)
    v1 = jnp.max(probs, axis=-1)
    i1 = jnp.min(jnp.where(probs >= v1[:, None], col, e), axis=-1)
    masked = jnp.where(col == i1[:, None], -jnp.inf, probs)
    v2 = jnp.max(masked, axis=-1)
    i2 = jnp.min(jnp.where(masked >= v2[:, None], col, e), axis=-1)
    s = v1 + v2
    wa = (v1 / s)[:, None]
    wb = (v2 / s)[:, None]
    comb = jnp.where(col == i1[:, None], wa, 0.0) + jnp.where(col == i2[:, None], wb, 0.0)
    ct_ref[...] = comb.T

    actf = (jnp.max(comb, axis=0, keepdims=True) > 0.0).astype(jnp.float32)
    r2 = jax.lax.broadcasted_iota(jnp.int32, (e, e), 0)
    c2 = jax.lax.broadcasted_iota(jnp.int32, (e, e), 1)
    tri = (r2 <= c2).astype(jnp.float32)
    cums = jnp.dot(actf, tri, preferred_element_type=jnp.float32)
    n = cums[0, e - 1]
    pos_t = (cums - 1.0).T
    match = (pos_t == c2.astype(jnp.float32)) & (actf.T > 0.0)
    erow = jax.lax.broadcasted_iota(jnp.int32, (1, e), 1).astype(jnp.float32)
    ids_sorted = jnp.dot(erow, match.astype(jnp.float32), preferred_element_type=jnp.float32)
    last = jnp.max(erow * actf - (1.0 - actf))
    ids_final = jnp.where(erow < n, ids_sorted, last)
    meta = jnp.concatenate([ids_final, jnp.full((1, e), n)], axis=1)
    meta_ref[...] = meta.astype(jnp.int32)


def _moe_body(meta_ref, x_ref, ct_ref, w1_ref, b1_ref, w2_ref, b2_ref, o_ref):
    i = pl.program_id(0)
    f = pl.program_id(1)
    n_e = ct_ref.shape[0]

    @pl.when(jnp.logical_and(i == 0, f == 0))
    def _init():
        o_ref[...] = jnp.zeros_like(o_ref)

    @pl.when(i < meta_ref[n_e])
    def _compute():
        x = x_ref[...]
        h = jnp.dot(x, w1_ref[0], preferred_element_type=jnp.float32)
        h = h + b1_ref[0]
        a = jax.nn.gelu(h)
        y = jnp.dot(a, w2_ref[0], preferred_element_type=jnp.float32)
        y = y + jnp.where(f == 0, 1.0, 0.0) * b2_ref[0]
        e = meta_ref[i]
        colw = ct_ref[e, :]
        o_ref[...] += colw[:, None] * y


def kernel(hidden_states, router_logits, w1, b1, w2, b2):
    t, d = hidden_states.shape
    n_e = router_logits.shape[1]
    ffn = w1.shape[2]
    n_fc = ffn // _F_BLK

    ct, meta = pl.pallas_call(
        _router_body,
        out_shape=[
            jax.ShapeDtypeStruct((n_e, t), jnp.float32),
            jax.ShapeDtypeStruct((1, 2 * n_e), jnp.int32),
        ],
    )(router_logits)
    meta = meta.reshape((2 * n_e,))

    b1_3 = b1[:, None, :]
    b2_3 = b2[:, None, :]

    def _f_eff(i, f, m):
        return jnp.where(i < m[n_e], f, n_fc - 1)

    grid_spec = pltpu.PrefetchScalarGridSpec(
        num_scalar_prefetch=1,
        grid=(n_e, n_fc),
        in_specs=[
            pl.BlockSpec((t, d), lambda i, f, m: (0, 0)),
            pl.BlockSpec((n_e, t), lambda i, f, m: (0, 0)),
            pl.BlockSpec((1, d, _F_BLK), lambda i, f, m: (m[i], 0, _f_eff(i, f, m))),
            pl.BlockSpec((1, 1, _F_BLK), lambda i, f, m: (m[i], 0, _f_eff(i, f, m))),
            pl.BlockSpec((1, _F_BLK, d), lambda i, f, m: (m[i], _f_eff(i, f, m), 0)),
            pl.BlockSpec((1, 1, d), lambda i, f, m: (m[i], 0, 0)),
        ],
        out_specs=pl.BlockSpec((t, d), lambda i, f, m: (0, 0)),
    )

    out = pl.pallas_call(
        _moe_body,
        grid_spec=grid_spec,
        out_shape=jax.ShapeDtypeStruct((t, d), jnp.float32),
        compiler_params=pltpu.CompilerParams(
            dimension_semantics=("arbitrary", "arbitrary"),
        ),
    )(meta, hidden_states, ct, w1, b1_3, w2, b2_3)
    return out
